# baseline (speedup 1.0000x reference)
import functools

import jax
import jax.numpy as jnp
from jax import lax
from jax.experimental import pallas as pl
from jax.experimental.pallas import tpu as pltpu
from jax.experimental.pallas import tpu_sc as plsc

_CH = 128
_NBUF = 4
_AHEAD = 2
_NC = 2
_NS = 16
_NW = _NC * _NS


def kernel(x, table):
    batch, seq = x.shape
    _, d_emb = table.shape
    n = batch * seq
    b_per_w = n // _NW
    nsteps = b_per_w // _CH
    assert b_per_w * _NW == n and nsteps * _CH == b_per_w
    assert nsteps % _NBUF == 0
    idx = x.reshape(n).astype(jnp.int32)

    mesh = plsc.VectorSubcoreMesh(core_axis_name="c", subcore_axis_name="s")

    @functools.partial(
        pl.kernel, mesh=mesh,
        out_type=jax.ShapeDtypeStruct((n, d_emb), table.dtype),
        scratch_types=(
            [pltpu.VMEM((b_per_w,), jnp.int32)]
            + [pltpu.VMEM((_CH, d_emb), jnp.float32)] * _NBUF
            + [pltpu.SemaphoreType.DMA] * (2 * _NBUF)
        ),
    )
    def gather_kernel(tab_hbm, idx_hbm, out_hbm, idx_v, *rest):
        bufs = rest[:_NBUF]
        gsem = rest[_NBUF:2 * _NBUF]
        osem = rest[2 * _NBUF:]
        wid = lax.axis_index("s") * _NC + lax.axis_index("c")
        base = wid * b_per_w
        pltpu.sync_copy(idx_hbm.at[pl.ds(base, b_per_w)], idx_v)

        def g_src(g):
            return tab_hbm.at[idx_v.at[pl.ds(g * _CH, _CH)]]

        def o_dst(g):
            return out_hbm.at[pl.ds(base + g * _CH, _CH)]

        for g in range(_AHEAD):
            pltpu.async_copy(g_src(g), bufs[g], gsem[g])

        @pl.loop(0, nsteps // _NBUF)
        def _(grp):
            for b in range(_NBUF):
                g = grp * _NBUF + b
                bn = (b + _AHEAD) % _NBUF
                nxt = g + _AHEAD
                pltpu.make_async_copy(g_src(g), bufs[b], gsem[b]).wait()
                pltpu.async_copy(bufs[b], o_dst(g), osem[b])

                @pl.when(nxt < nsteps)
                def _():
                    @pl.when(nxt >= _NBUF)
                    def _():
                        pltpu.make_async_copy(
                            bufs[bn], o_dst(nxt - _NBUF), osem[bn]).wait()

                    pltpu.async_copy(g_src(nxt), bufs[bn], gsem[bn])

        for k in range(_NBUF):
            g = nsteps - _NBUF + k
            pltpu.make_async_copy(bufs[g % _NBUF], o_dst(g),
                                  osem[g % _NBUF]).wait()

    out = gather_kernel(table, idx)
    return out.reshape(batch, seq, d_emb)


# speedup vs baseline: 1.0026x; 1.0026x over previous
import functools

import jax
import jax.numpy as jnp
from jax import lax
from jax.experimental import pallas as pl
from jax.experimental.pallas import tpu as pltpu
from jax.experimental.pallas import tpu_sc as plsc

_CH = 256
_NBUF = 2
_NC = 2
_NS = 16
_NW = _NC * _NS


def kernel(x, table):
    batch, seq = x.shape
    _, d_emb = table.shape
    n = batch * seq
    b_per_w = n // _NW
    nsteps = b_per_w // _CH
    assert b_per_w * _NW == n and nsteps * _CH == b_per_w
    assert nsteps % _NBUF == 0
    idx = x.reshape(n).astype(jnp.int32)

    mesh = plsc.VectorSubcoreMesh(core_axis_name="c", subcore_axis_name="s")

    @functools.partial(
        pl.kernel, mesh=mesh,
        out_type=jax.ShapeDtypeStruct((n, d_emb), table.dtype),
        scratch_types=[
            pltpu.VMEM((b_per_w,), jnp.int32),
            pltpu.VMEM((_CH, d_emb), jnp.float32),
            pltpu.VMEM((_CH, d_emb), jnp.float32),
            pltpu.SemaphoreType.DMA,
            pltpu.SemaphoreType.DMA,
            pltpu.SemaphoreType.DMA,
            pltpu.SemaphoreType.DMA,
        ],
    )
    def gather_kernel(tab_hbm, idx_hbm, out_hbm, idx_v, buf0, buf1,
                      gs0, gs1, os0, os1):
        wid = lax.axis_index("s") * _NC + lax.axis_index("c")
        base = wid * b_per_w
        pltpu.sync_copy(idx_hbm.at[pl.ds(base, b_per_w)], idx_v)

        bufs = (buf0, buf1)
        gsem = (gs0, gs1)
        osem = (os0, os1)

        def g_src(g):
            return tab_hbm.at[idx_v.at[pl.ds(g * _CH, _CH)]]

        def o_dst(g):
            return out_hbm.at[pl.ds(base + g * _CH, _CH)]

        for b in range(_NBUF):
            pltpu.async_copy(g_src(b), bufs[b], gsem[b])

        @pl.loop(0, nsteps // _NBUF)
        def _(grp):
            for b in range(_NBUF):
                g = grp * _NBUF + b
                pltpu.make_async_copy(g_src(g), bufs[b], gsem[b]).wait()
                pltpu.async_copy(bufs[b], o_dst(g), osem[b])
                pltpu.make_async_copy(bufs[b], o_dst(g), osem[b]).wait()
                nxt = g + _NBUF

                @pl.when(nxt < nsteps)
                def _():
                    pltpu.async_copy(g_src(nxt), bufs[b], gsem[b])

    out = gather_kernel(table, idx)
    return out.reshape(batch, seq, d_emb)
